# Initial kernel scaffold; baseline (speedup 1.0000x reference)
import jax
import jax.numpy as jnp
from jax.experimental import pallas as pl


def kernel(x_p, x_v, x_len, P_table, V_table, W, b):
    raise NotImplementedError("write your pallas kernel here")


# speedup vs baseline: 6.9505x; 6.9505x over previous
import functools

import jax
import jax.numpy as jnp
from jax import lax
from jax.experimental import pallas as pl
from jax.experimental.pallas import tpu as pltpu
from jax.experimental.pallas import tpu_sc as plsc

B, L = 4096, 200
D = 128
OUT = 512
NC, NS = 2, 16
NW = NC * NS
RPW = B // NW
C0, C1 = 96, 104
LANES = 16
DV = D // LANES


def _sc_embed_sums(x_p, x_v, P_table, V_table):
    mesh = plsc.VectorSubcoreMesh(
        core_axis_name="c", subcore_axis_name="s", num_cores=NC, num_subcores=NS
    )

    @functools.partial(
        pl.kernel,
        mesh=mesh,
        out_type=jax.ShapeDtypeStruct((B * 2 * D,), jnp.float32),
        scratch_types=[
            pltpu.VMEM((C0,), jnp.int32),
            pltpu.VMEM((C1,), jnp.int32),
            pltpu.VMEM((L, D), jnp.float32),
            pltpu.VMEM((L, D), jnp.float32),
            pltpu.VMEM((2 * D,), jnp.float32),
            pltpu.SemaphoreType.DMA,
        ],
    )
    def body(xp_hbm, xv_hbm, p_hbm, v_hbm, out_hbm, idxa, idxb, rows_p, rows_v, orow, sem):
        wid = lax.axis_index("s") * NC + lax.axis_index("c")
        base = wid * RPW

        def accum(buf):
            def red(j, accs):
                return tuple(a + buf[j, pl.ds(c * LANES, LANES)]
                             for c, a in enumerate(accs))
            init = tuple(jnp.zeros((LANES,), jnp.float32) for _ in range(DV))
            return lax.fori_loop(0, L, red, init)

        def gather(tab, idx_hbm, r, rows):
            off = pl.multiple_of(r * L, 8)
            pltpu.sync_copy(idx_hbm.at[pl.ds(off, C0)], idxa)
            pltpu.sync_copy(idx_hbm.at[pl.ds(off + C0, C1)], idxb)
            cp0 = pltpu.async_copy(tab.at[idxa], rows.at[pl.ds(0, C0)], sem)
            cp1 = pltpu.async_copy(tab.at[idxb], rows.at[pl.ds(C0, C1)], sem)
            cp0.wait()
            cp1.wait()

        def row(i, carry):
            r = base + i
            gather(p_hbm, xp_hbm, r, rows_p)
            p_acc = accum(rows_p)
            gather(v_hbm, xv_hbm, r, rows_v)
            v_acc = accum(rows_v)
            for c in range(DV):
                orow[pl.ds(c * LANES, LANES)] = p_acc[c]
                orow[pl.ds(D + c * LANES, LANES)] = v_acc[c]
            oof = pl.multiple_of(r * (2 * D), 8)
            pltpu.sync_copy(orow, out_hbm.at[pl.ds(oof, 2 * D)])
            return carry

        lax.fori_loop(0, RPW, row, 0)

    return body(x_p.reshape(B * L), x_v.reshape(B * L), P_table, V_table)


def _tc_head(sums, x_len, W, b2d):
    BT = 256

    def body(h_ref, len_ref, w_ref, b_ref, o_ref):
        h = h_ref[...] / len_ref[...].astype(jnp.float32)
        res = jnp.dot(h, w_ref[...], preferred_element_type=jnp.float32)
        res = res + b_ref[...]
        m = jnp.max(res, axis=-1, keepdims=True)
        e = res - m
        lse = jnp.log(jnp.sum(jnp.exp(e), axis=-1, keepdims=True))
        o_ref[...] = e - lse

    return pl.pallas_call(
        body,
        grid=(B // BT,),
        in_specs=[
            pl.BlockSpec((BT, 2 * D), lambda i: (i, 0)),
            pl.BlockSpec((BT, 1), lambda i: (i, 0)),
            pl.BlockSpec((2 * D, OUT), lambda i: (0, 0)),
            pl.BlockSpec((1, OUT), lambda i: (0, 0)),
        ],
        out_specs=pl.BlockSpec((BT, OUT), lambda i: (i, 0)),
        out_shape=jax.ShapeDtypeStruct((B, OUT), jnp.float32),
    )(sums, x_len, W, b2d)


@jax.jit
def kernel(x_p, x_v, x_len, P_table, V_table, W, b):
    sums = _sc_embed_sums(x_p, x_v, P_table, V_table).reshape(B, 2 * D)
    return _tc_head(sums, x_len, W, b.reshape(1, OUT))


# speedup vs baseline: 14.7372x; 2.1203x over previous
import functools

import jax
import jax.numpy as jnp
from jax import lax
from jax.experimental import pallas as pl
from jax.experimental.pallas import tpu as pltpu
from jax.experimental.pallas import tpu_sc as plsc

B, L = 4096, 200
D = 128
OUT = 512
NC, NS = 2, 16
NW = NC * NS
RPW = B // NW
C0, C1 = 96, 104
LANES = 16
DV = D // LANES
UNROLL = 4


def _sc_embed_sums(x_p, x_v, P_table, V_table):
    mesh = plsc.VectorSubcoreMesh(
        core_axis_name="c", subcore_axis_name="s", num_cores=NC, num_subcores=NS
    )

    @functools.partial(
        pl.kernel,
        mesh=mesh,
        out_type=jax.ShapeDtypeStruct((B * 2 * D,), jnp.float32),
        scratch_types=[
            pltpu.VMEM((RPW * L,), jnp.int32),
            pltpu.VMEM((RPW * L,), jnp.int32),
            pltpu.VMEM((L, D), jnp.float32),
            pltpu.VMEM((L, D), jnp.float32),
            pltpu.VMEM((2, 2 * D), jnp.float32),
            pltpu.SemaphoreType.DMA,
            pltpu.SemaphoreType.DMA,
            pltpu.SemaphoreType.DMA,
        ],
    )
    def body(xp_hbm, xv_hbm, p_hbm, v_hbm, out_hbm,
             idxp, idxv, buf_p, buf_v, orow, psem, vsem, osem):
        wid = lax.axis_index("s") * NC + lax.axis_index("c")
        base = wid * RPW

        pltpu.sync_copy(xp_hbm.at[pl.ds(base * L, RPW * L)], idxp)
        pltpu.sync_copy(xv_hbm.at[pl.ds(base * L, RPW * L)], idxv)

        def fire(tab, idx, i, buf, sem):
            o = pl.multiple_of(i * L, 8)
            pltpu.async_copy(tab.at[idx.at[pl.ds(o, C0)]],
                             buf.at[pl.ds(0, C0)], sem)
            pltpu.async_copy(tab.at[idx.at[pl.ds(o + C0, C1)]],
                             buf.at[pl.ds(C0, C1)], sem)

        def drain(tab, buf, sem):
            pltpu.make_async_copy(tab.at[pl.ds(0, L)], buf, sem).wait()

        def drain_out():
            pltpu.make_async_copy(orow.at[0], out_hbm.at[pl.ds(0, 2 * D)],
                                  osem).wait()

        def accum(buf):
            def red(j4, accs):
                j = j4 * UNROLL
                for u in range(UNROLL):
                    accs = tuple(a + buf[j + u, pl.ds(c * LANES, LANES)]
                                 for c, a in enumerate(accs))
                return accs
            init = tuple(jnp.zeros((LANES,), jnp.float32) for _ in range(DV))
            return lax.fori_loop(0, L // UNROLL, red, init)

        fire(p_hbm, idxp, 0, buf_p, psem)

        def row(i, carry):
            slot = i % 2
            fire(v_hbm, idxv, i, buf_v, vsem)
            drain(p_hbm, buf_p, psem)
            p_acc = accum(buf_p)
            nxt = jnp.minimum(i + 1, RPW - 1)
            fire(p_hbm, idxp, nxt, buf_p, psem)
            drain(v_hbm, buf_v, vsem)
            v_acc = accum(buf_v)
            for c in range(DV):
                orow[slot, pl.ds(c * LANES, LANES)] = p_acc[c]
                orow[slot, pl.ds(D + c * LANES, LANES)] = v_acc[c]
            oof = pl.multiple_of((base + i) * (2 * D), 8)
            pltpu.async_copy(orow.at[slot], out_hbm.at[pl.ds(oof, 2 * D)], osem)

            @pl.when(i >= 2)
            def _():
                drain_out()
            return carry

        lax.fori_loop(0, RPW, row, 0)
        drain(p_hbm, buf_p, psem)
        drain_out()
        drain_out()

    return body(x_p.reshape(B * L), x_v.reshape(B * L), P_table, V_table)


def _tc_head(sums, x_len, W, b2d):
    BT = 256

    def body(h_ref, len_ref, w_ref, b_ref, o_ref):
        h = h_ref[...] / len_ref[...].astype(jnp.float32)
        res = jnp.dot(h, w_ref[...], preferred_element_type=jnp.float32)
        res = res + b_ref[...]
        m = jnp.max(res, axis=-1, keepdims=True)
        e = res - m
        lse = jnp.log(jnp.sum(jnp.exp(e), axis=-1, keepdims=True))
        o_ref[...] = e - lse

    return pl.pallas_call(
        body,
        grid=(B // BT,),
        in_specs=[
            pl.BlockSpec((BT, 2 * D), lambda i: (i, 0)),
            pl.BlockSpec((BT, 1), lambda i: (i, 0)),
            pl.BlockSpec((2 * D, OUT), lambda i: (0, 0)),
            pl.BlockSpec((1, OUT), lambda i: (0, 0)),
        ],
        out_specs=pl.BlockSpec((BT, OUT), lambda i: (i, 0)),
        out_shape=jax.ShapeDtypeStruct((B, OUT), jnp.float32),
    )(sums, x_len, W, b2d)


@jax.jit
def kernel(x_p, x_v, x_len, P_table, V_table, W, b):
    sums = _sc_embed_sums(x_p, x_v, P_table, V_table).reshape(B, 2 * D)
    return _tc_head(sums, x_len, W, b.reshape(1, OUT))


# speedup vs baseline: 19.9472x; 1.3535x over previous
import functools

import jax
import jax.numpy as jnp
from jax import lax
from jax.experimental import pallas as pl
from jax.experimental.pallas import tpu as pltpu
from jax.experimental.pallas import tpu_sc as plsc

B, L = 4096, 200
D = 128
OUT = 512
PV = 1024
NC, NS = 2, 16
NW = NC * NS
RPW = B // NW
RPC = B // NC
NPASS = 4
PASS_ROWS = RPC // NPASS
BLK = RPW // NPASS
TILE_REGION = PASS_ROWS * PV // NS
C0, C1 = 96, 104
LANES = 16
DV = D // LANES
UNROLL = 4
NCHUNK = (RPW * L) // 128
CPP = NCHUNK // NPASS
ZCH = 8192
SDEPTH = 8


def _sc_embed(offs3d, x_v_flat, V_table):
    mesh = plsc.VectorSubcoreMesh(
        core_axis_name="c", subcore_axis_name="s", num_cores=NC, num_subcores=NS
    )

    @functools.partial(
        pl.kernel,
        mesh=mesh,
        out_type=(
            jax.ShapeDtypeStruct((B * D,), jnp.float32),
            jax.ShapeDtypeStruct((B * PV,), jnp.float32),
        ),
        scratch_types=[
            pltpu.VMEM((NCHUNK, 128), jnp.int32),
            pltpu.VMEM((RPW * L,), jnp.int32),
            pltpu.VMEM((C0, D), jnp.float32),
            pltpu.VMEM((C1, D), jnp.float32),
            pltpu.VMEM((ZCH,), jnp.float32),
            pltpu.VMEM((128,), jnp.float32),
            pltpu.VMEM((2, D), jnp.float32),
            pltpu.VMEM_SHARED((PASS_ROWS * PV,), jnp.float32),
            pltpu.SemaphoreType.DMA,
            pltpu.SemaphoreType.DMA,
            pltpu.SemaphoreType.DMA,
            pltpu.SemaphoreType.DMA,
            pltpu.SemaphoreType.DMA,
            pltpu.SemaphoreType.DMA,
        ],
    )
    def body(offs_hbm, xv_hbm, v_hbm, vout_hbm, cnt_hbm,
             offsv, idxv, bufa, bufb, zblk, ones, orow, csp,
             zsem, ssem, csem, vsem_a, vsem_b, osem):
        c = lax.axis_index("c")
        s = lax.axis_index("s")
        wid = c * NS + s
        base = wid * RPW

        pltpu.sync_copy(offs_hbm.at[wid], offsv)
        pltpu.sync_copy(xv_hbm.at[pl.ds(base * L, RPW * L)], idxv)

        def fill(j, carry):
            zblk[pl.ds(j * LANES, LANES)] = jnp.zeros((LANES,), jnp.float32)
            return carry
        lax.fori_loop(0, ZCH // LANES, fill, 0)
        for k in range(128 // LANES):
            ones[pl.ds(k * LANES, LANES)] = jnp.ones((LANES,), jnp.float32)

        my_region = pl.multiple_of(s * TILE_REGION, 8)

        def drain_scatter():
            pltpu.make_async_copy(ones, csp.at[pl.ds(0, 128)], ssem).wait()

        def drain_copyout():
            pltpu.make_async_copy(csp.at[pl.ds(0, TILE_REGION)],
                                  cnt_hbm.at[pl.ds(0, TILE_REGION)],
                                  csem).wait()

        for p in range(NPASS):
            if p > 0:
                drain_copyout()
            for k in range(TILE_REGION // ZCH):
                pltpu.async_copy(zblk, csp.at[pl.ds(my_region + k * ZCH, ZCH)],
                                 zsem)
            for k in range(TILE_REGION // ZCH):
                pltpu.make_async_copy(zblk, csp.at[pl.ds(0, ZCH)], zsem).wait()
            plsc.subcore_barrier()

            def scat(j, carry):
                pltpu.async_copy(ones, csp.at[offsv.at[p * CPP + j]], ssem,
                                 add=True)

                @pl.when(j >= SDEPTH)
                def _():
                    drain_scatter()
                return carry
            lax.fori_loop(0, CPP, scat, 0)
            for _ in range(SDEPTH):
                drain_scatter()
            plsc.subcore_barrier()

            dst = pl.multiple_of((c * RPC + (NPASS * s + p) * BLK) * PV, 8)
            pltpu.async_copy(csp.at[pl.ds(my_region, TILE_REGION)],
                             cnt_hbm.at[pl.ds(dst, TILE_REGION)], csem)

        def fire_a(i):
            o = pl.multiple_of(i * L, 8)
            pltpu.async_copy(v_hbm.at[idxv.at[pl.ds(o, C0)]], bufa, vsem_a)

        def fire_b(i):
            o = pl.multiple_of(i * L + C0, 8)
            pltpu.async_copy(v_hbm.at[idxv.at[pl.ds(o, C1)]], bufb, vsem_b)

        def drain_a():
            pltpu.make_async_copy(v_hbm.at[pl.ds(0, C0)], bufa, vsem_a).wait()

        def drain_b():
            pltpu.make_async_copy(v_hbm.at[pl.ds(0, C1)], bufb, vsem_b).wait()

        def drain_out():
            pltpu.make_async_copy(orow.at[0], vout_hbm.at[pl.ds(0, D)],
                                  osem).wait()

        def accum(buf, n, init):
            def red(j4, accs):
                j = j4 * UNROLL
                for u in range(UNROLL):
                    accs = tuple(a + buf[j + u, pl.ds(cc * LANES, LANES)]
                                 for cc, a in enumerate(accs))
                return accs
            return lax.fori_loop(0, n // UNROLL, red, init)

        zeros8 = tuple(jnp.zeros((LANES,), jnp.float32) for _ in range(DV))

        def finish(i, slot, accs):
            for cc in range(DV):
                orow[slot, pl.ds(cc * LANES, LANES)] = accs[cc]
            oof = pl.multiple_of((base + i) * D, 8)
            pltpu.async_copy(orow.at[slot], vout_hbm.at[pl.ds(oof, D)], osem)

        def do_row(r, slot, nxt):
            drain_a()
            acc = accum(bufa, C0, zeros8)
            fire_a(nxt)
            drain_b()
            acc = accum(bufb, C1, acc)
            fire_b(nxt)
            finish(r, slot, acc)

        fire_a(0)
        fire_b(0)

        def pair(i2, carry):
            r0 = 2 * i2
            do_row(r0, 0, r0 + 1)
            do_row(r0 + 1, 1, jnp.minimum(r0 + 2, RPW - 1))

            @pl.when(i2 >= 1)
            def _():
                drain_out()
                drain_out()
            return carry

        lax.fori_loop(0, RPW // 2, pair, 0)
        drain_a()
        drain_b()
        drain_out()
        drain_out()
        drain_copyout()

    return body(offs3d, x_v_flat, V_table)


def _tc_head(counts, vsums, x_len, P_pad, W, b2d):
    BT = 256

    def body(c_ref, v_ref, len_ref, p_ref, w_ref, b_ref, o_ref):
        p_sum = jnp.dot(c_ref[...], p_ref[...],
                        preferred_element_type=jnp.float32)
        h = jnp.concatenate([p_sum, v_ref[...]], axis=1)
        h = h / len_ref[...].astype(jnp.float32)
        res = jnp.dot(h, w_ref[...], preferred_element_type=jnp.float32)
        res = res + b_ref[...]
        m = jnp.max(res, axis=-1, keepdims=True)
        e = res - m
        lse = jnp.log(jnp.sum(jnp.exp(e), axis=-1, keepdims=True))
        o_ref[...] = e - lse

    return pl.pallas_call(
        body,
        grid=(B // BT,),
        in_specs=[
            pl.BlockSpec((BT, PV), lambda i: (i, 0)),
            pl.BlockSpec((BT, D), lambda i: (i, 0)),
            pl.BlockSpec((BT, 1), lambda i: (i, 0)),
            pl.BlockSpec((PV, D), lambda i: (0, 0)),
            pl.BlockSpec((2 * D, OUT), lambda i: (0, 0)),
            pl.BlockSpec((1, OUT), lambda i: (0, 0)),
        ],
        out_specs=pl.BlockSpec((BT, OUT), lambda i: (i, 0)),
        out_shape=jax.ShapeDtypeStruct((B, OUT), jnp.float32),
    )(counts, vsums, x_len, P_pad, W, b2d)


@jax.jit
def kernel(x_p, x_v, x_len, P_table, V_table, W, b):
    r = jnp.arange(B, dtype=jnp.int32) % RPC
    blk = r // BLK
    slot_row = (blk // NPASS) * BLK + (r % BLK)
    offs = x_p + slot_row[:, None] * PV
    offs3d = offs.reshape(NW, NCHUNK, 128)

    vsums, counts = _sc_embed(offs3d, x_v.reshape(B * L), V_table)
    P_pad = jnp.zeros((PV, D), jnp.float32).at[:1000].set(P_table)
    return _tc_head(counts.reshape(B, PV), vsums.reshape(B, D),
                    x_len, P_pad, W, b.reshape(1, OUT))


# speedup vs baseline: 22.0702x; 1.1064x over previous
import functools

import jax
import jax.numpy as jnp
from jax import lax
from jax.experimental import pallas as pl
from jax.experimental.pallas import tpu as pltpu
from jax.experimental.pallas import tpu_sc as plsc

B, L = 4096, 200
D = 128
OUT = 512
PVOC = 1000
NC, NS = 2, 16
NW = NC * NS
RPW = B // NW
C0, C1 = 96, 104
LANES = 16
DV = D // LANES
UNROLL = 4
G = 8
NG = RPW // G


def _sc_embed(x_p, x_v, P_table, V_table):
    mesh = plsc.VectorSubcoreMesh(
        core_axis_name="c", subcore_axis_name="s", num_cores=NC, num_subcores=NS
    )

    @functools.partial(
        pl.kernel,
        mesh=mesh,
        out_type=jax.ShapeDtypeStruct((B, 2 * D), jnp.float32),
        scratch_types=[
            pltpu.VMEM((RPW * L,), jnp.int32),
            pltpu.VMEM((RPW * L,), jnp.int32),
            pltpu.VMEM((C0, D), jnp.float32),
            pltpu.VMEM((C1, D), jnp.float32),
            pltpu.VMEM((C0, D), jnp.float32),
            pltpu.VMEM((C1, D), jnp.float32),
            pltpu.VMEM((2, G, 2 * D), jnp.float32),
            pltpu.VMEM_SHARED((PVOC, D), jnp.float32),
            pltpu.SemaphoreType.DMA,
            pltpu.SemaphoreType.DMA,
            pltpu.SemaphoreType.DMA,
            pltpu.SemaphoreType.DMA,
            pltpu.SemaphoreType.DMA,
        ],
    )
    def body(xp_hbm, xv_hbm, p_hbm, v_hbm, out_hbm,
             idxp, idxv, bpa, bpb, bva, bvb, ostage, psp,
             pas, pbs, vas, vbs, osem):
        c = lax.axis_index("c")
        s = lax.axis_index("s")
        wid = c * NS + s
        base = wid * RPW

        boff = pl.multiple_of(base * L, 8)
        pltpu.sync_copy(xp_hbm.at[pl.ds(boff, RPW * L)], idxp)
        pltpu.sync_copy(xv_hbm.at[pl.ds(boff, RPW * L)], idxv)

        po = pl.multiple_of(s * 64, 8)

        @pl.when(s < NS - 1)
        def _():
            pltpu.sync_copy(p_hbm.at[pl.ds(po, 64)], psp.at[pl.ds(po, 64)])

        @pl.when(s == NS - 1)
        def _():
            pltpu.sync_copy(p_hbm.at[pl.ds(960, 40)], psp.at[pl.ds(960, 40)])
        plsc.subcore_barrier()

        def fire(tab, idx, i, buf, sem, lo, n):
            o = pl.multiple_of(i * L + lo, 8)
            pltpu.async_copy(tab.at[idx.at[pl.ds(o, n)]], buf, sem)

        def drain(tab, buf, sem):
            pltpu.make_async_copy(tab.at[pl.ds(0, buf.shape[0])], buf,
                                  sem).wait()

        def drain_out():
            pltpu.make_async_copy(ostage.at[0],
                                  out_hbm.at[pl.ds(0, G)], osem).wait()

        def accum(buf, n, init):
            def red(j4, accs):
                j = j4 * UNROLL
                for u in range(UNROLL):
                    accs = tuple(a + buf[j + u, pl.ds(cc * LANES, LANES)]
                                 for cc, a in enumerate(accs))
                return accs
            return lax.fori_loop(0, n // UNROLL, red, init)

        zeros8 = tuple(jnp.zeros((LANES,), jnp.float32) for _ in range(DV))

        def fire_all(i):
            fire(psp, idxp, i, bpa, pas, 0, C0)
            fire(psp, idxp, i, bpb, pbs, C0, C1)
            fire(v_hbm, idxv, i, bva, vas, 0, C0)
            fire(v_hbm, idxv, i, bvb, vbs, C0, C1)

        fire_all(0)

        def group(g, carry):
            gp = g % 2
            for k in range(G):
                i = g * G + k
                nxt = jnp.minimum(i + 1, RPW - 1)
                drain(psp, bpa, pas)
                pacc = accum(bpa, C0, zeros8)
                fire(psp, idxp, nxt, bpa, pas, 0, C0)
                drain(psp, bpb, pbs)
                pacc = accum(bpb, C1, pacc)
                fire(psp, idxp, nxt, bpb, pbs, C0, C1)
                drain(v_hbm, bva, vas)
                vacc = accum(bva, C0, zeros8)
                fire(v_hbm, idxv, nxt, bva, vas, 0, C0)
                drain(v_hbm, bvb, vbs)
                vacc = accum(bvb, C1, vacc)
                fire(v_hbm, idxv, nxt, bvb, vbs, C0, C1)
                for cc in range(DV):
                    ostage[gp, k, pl.ds(cc * LANES, LANES)] = pacc[cc]
                    ostage[gp, k, pl.ds(D + cc * LANES, LANES)] = vacc[cc]
            oof = pl.multiple_of(base + g * G, 8)
            pltpu.async_copy(ostage.at[gp], out_hbm.at[pl.ds(oof, G)], osem)

            @pl.when(g >= 2)
            def _():
                drain_out()
            return carry

        lax.fori_loop(0, NG, group, 0)
        drain(psp, bpa, pas)
        drain(psp, bpb, pbs)
        drain(v_hbm, bva, vas)
        drain(v_hbm, bvb, vbs)
        drain_out()
        drain_out()

    return body(x_p.reshape(B * L), x_v.reshape(B * L), P_table, V_table)


def _tc_head(sums, x_len, W, b2d):
    BT = 256

    def body(h_ref, len_ref, w_ref, b_ref, o_ref):
        h = h_ref[...] / len_ref[...].astype(jnp.float32)
        res = jnp.dot(h, w_ref[...], preferred_element_type=jnp.float32)
        res = res + b_ref[...]
        m = jnp.max(res, axis=-1, keepdims=True)
        e = res - m
        lse = jnp.log(jnp.sum(jnp.exp(e), axis=-1, keepdims=True))
        o_ref[...] = e - lse

    return pl.pallas_call(
        body,
        grid=(B // BT,),
        in_specs=[
            pl.BlockSpec((BT, 2 * D), lambda i: (i, 0)),
            pl.BlockSpec((BT, 1), lambda i: (i, 0)),
            pl.BlockSpec((2 * D, OUT), lambda i: (0, 0)),
            pl.BlockSpec((1, OUT), lambda i: (0, 0)),
        ],
        out_specs=pl.BlockSpec((BT, OUT), lambda i: (i, 0)),
        out_shape=jax.ShapeDtypeStruct((B, OUT), jnp.float32),
    )(sums, x_len, W, b2d)


@jax.jit
def kernel(x_p, x_v, x_len, P_table, V_table, W, b):
    sums = _sc_embed(x_p, x_v, P_table, V_table)
    return _tc_head(sums, x_len, W, b.reshape(1, OUT))
